# Initial kernel scaffold; baseline (speedup 1.0000x reference)
import jax
import jax.numpy as jnp
from jax.experimental import pallas as pl


def kernel(in_feat, edge_index, W1_pool, b1_pool, W1_self, W1_eve, pw1, bias1, W2_pool, b2_pool, W2_self, W2_eve, pw2, bias2):
    raise NotImplementedError("write your pallas kernel here")


# speedup vs baseline: 1.0829x; 1.0829x over previous
import functools

import jax
import jax.numpy as jnp
from jax.experimental import pallas as pl
from jax.experimental.pallas import tpu as pltpu

N = 10000
D = 256

_BLK = 1000


def _pool_body(x_ref, w_ref, b_ref, o_ref):
    acc = jnp.dot(x_ref[...], w_ref[...].T, preferred_element_type=jnp.float32)
    o_ref[...] = jnp.maximum(acc + b_ref[...][None, :], 0.0)


def _pool_matmul(x, w, b):
    return pl.pallas_call(
        _pool_body,
        grid=(N // _BLK,),
        in_specs=[
            pl.BlockSpec((_BLK, D), lambda i: (i, 0)),
            pl.BlockSpec((D, D), lambda i: (0, 0)),
            pl.BlockSpec((D,), lambda i: (0,)),
        ],
        out_specs=pl.BlockSpec((_BLK, D), lambda i: (i, 0)),
        out_shape=jax.ShapeDtypeStruct((N, D), jnp.float32),
    )(x, w, b)


def _combine_body(x_ref, mx_ref, mn_ref, ws_ref, we0_ref, we1_ref, b_ref,
                  o_ref, *, relu):
    mn = mn_ref[...]
    mn = jnp.where(mn >= 1e30, 0.0, mn)
    acc = jnp.dot(x_ref[...], ws_ref[...].T, preferred_element_type=jnp.float32)
    acc += jnp.dot(mx_ref[...], we0_ref[...].T, preferred_element_type=jnp.float32)
    acc += jnp.dot(mn, we1_ref[...].T, preferred_element_type=jnp.float32)
    acc += b_ref[...][None, :]
    if relu:
        acc = jnp.maximum(acc, 0.0)
    o_ref[...] = acc


def _combine(x, mx, mn, w_self, we0, we1, bias, relu):
    return pl.pallas_call(
        functools.partial(_combine_body, relu=relu),
        grid=(N // _BLK,),
        in_specs=[
            pl.BlockSpec((_BLK, D), lambda i: (i, 0)),
            pl.BlockSpec((_BLK, D), lambda i: (i, 0)),
            pl.BlockSpec((_BLK, D), lambda i: (i, 0)),
            pl.BlockSpec((D, D), lambda i: (0, 0)),
            pl.BlockSpec((D, D), lambda i: (0, 0)),
            pl.BlockSpec((D, D), lambda i: (0, 0)),
            pl.BlockSpec((D,), lambda i: (0,)),
        ],
        out_specs=pl.BlockSpec((_BLK, D), lambda i: (i, 0)),
        out_shape=jax.ShapeDtypeStruct((N, D), jnp.float32),
    )(x, mx, mn, w_self, we0, we1, bias)


def _seg_minmax(h, src, dst):
    m = jnp.take(h, src, axis=0)
    mx = jax.ops.segment_max(m, dst, num_segments=N)
    mn = jax.ops.segment_min(m, dst, num_segments=N)
    mx = jnp.maximum(mx, 0.0)
    return mx, mn


def kernel(in_feat, edge_index, W1_pool, b1_pool, W1_self, W1_eve, pw1, bias1,
           W2_pool, b2_pool, W2_self, W2_eve, pw2, bias2):
    src = edge_index[0].astype(jnp.int32)
    dst = edge_index[1].astype(jnp.int32)

    def layer(x, W_pool, b_pool, W_self, W_eve, pw, bias, relu):
        h = _pool_matmul(x, W_pool, b_pool)
        mx, mn = _seg_minmax(h, src, dst)
        we0 = pw[0] * W_eve
        we1 = pw[1] * W_eve
        return _combine(x, mx, mn, W_self, we0, we1, bias, relu)

    h1 = layer(in_feat, W1_pool, b1_pool, W1_self, W1_eve, pw1, bias1, True)
    out = layer(h1, W2_pool, b2_pool, W2_self, W2_eve, pw2, bias2, False)
    return out


# speedup vs baseline: 6.1342x; 5.6646x over previous
import functools

import jax
import jax.numpy as jnp
from jax import lax
from jax.experimental import pallas as pl
from jax.experimental.pallas import tpu as pltpu
from jax.experimental.pallas import tpu_sc as plsc

N = 10000
E = 160000
D = 256

NT = 32
EC = E // NT
RNG = 320
ND = NT * RNG
GR = 160
K = 128
NCHUNK = (EC + K - 1) // K
ECPAD = NCHUNK * K
BINCAP = E + 4096

_BLK = 1000

_mesh = plsc.VectorSubcoreMesh(core_axis_name="c", subcore_axis_name="s")


def _wid():
    return lax.axis_index("s") * 2 + lax.axis_index("c")


def _sread(ref, i):
    return ref[pl.ds(i, 16)][0]


def _runinfo(sd):
    iota = lax.iota(jnp.int32, 16)
    prev = sd.at[jnp.maximum(iota - 1, 0)].get(mode="promise_in_bounds")
    first = (sd != prev) | (iota == 0)
    runstart = plsc.cummax(jnp.where(first, iota, 0))
    rank = iota - runstart
    nfirst = first.astype(jnp.int32).at[
        jnp.minimum(iota + 1, 15)].get(mode="promise_in_bounds")
    last = (iota == 15) | (nfirst == 1)
    return rank, last


def _pool_body(x_ref, w_ref, b_ref, o_ref):
    acc = jnp.dot(x_ref[...], w_ref[...].T, preferred_element_type=jnp.float32)
    o_ref[...] = jnp.maximum(acc + b_ref[...][None, :], 0.0)


def _pool_matmul(x, w, b):
    return pl.pallas_call(
        _pool_body,
        grid=(N // _BLK,),
        in_specs=[
            pl.BlockSpec((_BLK, D), lambda i: (i, 0)),
            pl.BlockSpec((D, D), lambda i: (0, 0)),
            pl.BlockSpec((D,), lambda i: (0,)),
        ],
        out_specs=pl.BlockSpec((_BLK, D), lambda i: (i, 0)),
        out_shape=jax.ShapeDtypeStruct((N, D), jnp.float32),
    )(x, w, b)


def _combine_body(x_ref, mx_ref, mn_ref, ws_ref, we0_ref, we1_ref, b_ref,
                  o_ref, *, relu):
    mn = mn_ref[...]
    mn = jnp.where(mn >= 1e30, 0.0, mn)
    acc = jnp.dot(x_ref[...], ws_ref[...].T, preferred_element_type=jnp.float32)
    acc += jnp.dot(mx_ref[...], we0_ref[...].T, preferred_element_type=jnp.float32)
    acc += jnp.dot(mn, we1_ref[...].T, preferred_element_type=jnp.float32)
    acc += b_ref[...][None, :]
    if relu:
        acc = jnp.maximum(acc, 0.0)
    o_ref[...] = acc


def _combine(x, mx, mn, w_self, we0, we1, bias, relu):
    return pl.pallas_call(
        functools.partial(_combine_body, relu=relu),
        grid=(N // _BLK,),
        in_specs=[
            pl.BlockSpec((_BLK, D), lambda i: (i, 0)),
            pl.BlockSpec((_BLK, D), lambda i: (i, 0)),
            pl.BlockSpec((_BLK, D), lambda i: (i, 0)),
            pl.BlockSpec((D, D), lambda i: (0, 0)),
            pl.BlockSpec((D, D), lambda i: (0, 0)),
            pl.BlockSpec((D, D), lambda i: (0, 0)),
            pl.BlockSpec((D,), lambda i: (0,)),
        ],
        out_specs=pl.BlockSpec((_BLK, D), lambda i: (i, 0)),
        out_shape=jax.ShapeDtypeStruct((N, D), jnp.float32),
    )(x, mx, mn, w_self, we0, we1, bias)


def _b1_body(dst_hbm, hist_hbm, dst_v, hist_v):
    t = _wid()
    zero = jnp.zeros((16,), jnp.int32)
    iota = lax.iota(jnp.int32, 16)

    def zinit(i, c):
        hist_v[pl.ds(i * 16, 16)] = zero
        return c
    lax.fori_loop(0, ND // 16, zinit, 0)

    pltpu.sync_copy(dst_hbm.at[pl.ds(pl.multiple_of(t * EC, 8), EC)],
                    dst_v.at[pl.ds(0, EC)])

    def cnt(i, c):
        base = i * 16
        d = dst_v[pl.ds(base, 16)]
        valid = (base + iota) < EC
        d = jnp.where(valid, d, ND - 1)
        sd, _ = plsc.sort_key_val(d, d)
        rank, last = _runinfo(sd)
        old = plsc.load_gather(hist_v, [sd])
        plsc.store_scatter(hist_v, [sd], old + rank + 1,
                           mask=last & (sd != ND - 1))
        return c
    lax.fori_loop(0, (EC + 15) // 16, cnt, 0)

    pltpu.sync_copy(hist_v, hist_hbm.at[pl.ds(pl.multiple_of(t * ND, 8), ND)])


_b1 = functools.partial(
    pl.kernel,
    compiler_params=pltpu.CompilerParams(needs_layout_passes=False),
    out_type=jax.ShapeDtypeStruct((NT * ND,), jnp.int32),
    mesh=_mesh,
    scratch_types=[
        pltpu.VMEM((ECPAD,), jnp.int32),
        pltpu.VMEM((ND,), jnp.int32),
    ],
)(_b1_body)


def _b2_body(hist_hbm, offs_hbm, rtot_hbm, cols, pre, sloc, rt_row):
    u = _wid()
    lo = u * RNG

    def load_t(t, c):
        pltpu.sync_copy(hist_hbm.at[pl.ds(pl.multiple_of(t * ND + lo, 8), RNG)],
                        cols.at[pl.ds(pl.multiple_of(t * RNG, 8), RNG)])
        return c
    lax.fori_loop(0, NT, load_t, 0)

    for j in range(RNG // 16):
        sl = pl.ds(j * 16, 16)

        def body_t(t, acc, j=j):
            pre[pl.ds(t * RNG + j * 16, 16)] = acc
            return acc + cols[pl.ds(t * RNG + j * 16, 16)]
        tot = lax.fori_loop(0, NT, body_t, jnp.zeros((16,), jnp.int32))
        sloc[sl] = tot

    carry = jnp.int32(0)
    for j in range(RNG // 16):
        sl = pl.ds(j * 16, 16)
        v = sloc[sl]
        incl = plsc.cumsum(v)
        sloc[sl] = incl - v + carry
        carry = carry + jnp.sum(v)

    def out_t(t, c):
        for j in range(RNG // 16):
            sl = pl.ds(j * 16, 16)
            o = pl.ds(t * RNG + j * 16, 16)
            pre[o] = pre[o] + sloc[sl]
        pltpu.sync_copy(pre.at[pl.ds(pl.multiple_of(t * RNG, 8), RNG)],
                        offs_hbm.at[pl.ds(pl.multiple_of(t * ND + lo, 8), RNG)])
        return c
    lax.fori_loop(0, NT, out_t, 0)

    rt_row[...] = jnp.zeros((16,), jnp.int32) + carry
    pltpu.sync_copy(rt_row, rtot_hbm.at[pl.ds(pl.multiple_of(u * 16, 8), 16)])


_b2 = functools.partial(
    pl.kernel,
    compiler_params=pltpu.CompilerParams(needs_layout_passes=False),
    out_type=(
        jax.ShapeDtypeStruct((NT * ND,), jnp.int32),
        jax.ShapeDtypeStruct((NT * 16,), jnp.int32),
    ),
    mesh=_mesh,
    scratch_types=[
        pltpu.VMEM((NT * RNG,), jnp.int32),
        pltpu.VMEM((NT * RNG,), jnp.int32),
        pltpu.VMEM((RNG,), jnp.int32),
        pltpu.VMEM((16,), jnp.int32),
    ],
)(_b2_body)


def _b3_body(src_hbm, dst_hbm, offs_hbm, rtot_hbm, binned_hbm, sg_hbm,
             offs_v, rt_v, base_v, src_v, dst_v, pos_v, dat_v, zpad_v, sem):
    t = _wid()
    iota = lax.iota(jnp.int32, 16)
    pltpu.sync_copy(offs_hbm.at[pl.ds(pl.multiple_of(t * ND, 8), ND)],
                    offs_v.at[pl.ds(0, ND)])
    pltpu.sync_copy(rtot_hbm, rt_v)

    t0 = plsc.load_gather(rt_v, [iota * 16])
    t1 = plsc.load_gather(rt_v, [(iota + 16) * 16])
    base_v[pl.ds(0, 16)] = plsc.cumsum(t0) - t0
    base_v[pl.ds(16, 16)] = plsc.cumsum(t1) - t1 + jnp.sum(t0)

    def addb(u, c):
        b = _sread(base_v, u)
        bv = jnp.zeros((16,), jnp.int32) + b
        for j in range(RNG // 16):
            sl = pl.ds(u * RNG + j * 16, 16)
            offs_v[sl] = offs_v[sl] + bv
        return c
    lax.fori_loop(0, NT, addb, 0)

    @pl.when(t == 0)
    def _():
        offs_v[pl.ds(ND, 16)] = jnp.zeros((16,), jnp.int32) + E
        pltpu.sync_copy(offs_v, sg_hbm)

    pltpu.sync_copy(src_hbm.at[pl.ds(pl.multiple_of(t * EC, 8), EC)],
                    src_v.at[pl.ds(0, EC)])
    pltpu.sync_copy(dst_hbm.at[pl.ds(pl.multiple_of(t * EC, 8), EC)],
                    dst_v.at[pl.ds(0, EC)])

    def emit_ch(ch, c):
        for jj in range(K // 16):
            base = (ch * (K // 16) + jj) * 16
            d = dst_v[pl.ds(base, 16)]
            s = src_v[pl.ds(base, 16)]
            valid = (base + iota) < EC
            d = jnp.where(valid, d, ND - 1)
            sd, ss = plsc.sort_key_val(d, s)
            rank, last = _runinfo(sd)
            inval = sd == ND - 1
            old = plsc.load_gather(offs_v, [sd])
            plsc.store_scatter(offs_v, [sd], old + rank + 1,
                               mask=last & ~inval)
            pos = jnp.where(inval, E + t * K + jj * 16 + iota, old + rank)
            pos_v[ch, pl.ds(jj * 16, 16)] = pos
            dat_v[ch, pl.ds(jj * 16, 16)] = jnp.where(inval, 0, ss)
        return c
    lax.fori_loop(0, NCHUNK, emit_ch, 0)

    for jj in range(K // 16):
        zpad_v[pl.ds(jj * 16, 16)] = jnp.zeros((16,), jnp.int32)
    pltpu.sync_copy(zpad_v, binned_hbm.at[pl.ds(pl.multiple_of(E + t * K, 8), K)])

    descs = [
        pltpu.async_copy(dat_v.at[ch], binned_hbm.at[pos_v.at[ch]], sem)
        for ch in range(NCHUNK)
    ]
    for dsc in descs:
        dsc.wait()


_b3 = functools.partial(
    pl.kernel,
    compiler_params=pltpu.CompilerParams(needs_layout_passes=False),
    out_type=(
        jax.ShapeDtypeStruct((BINCAP,), jnp.int32),
        jax.ShapeDtypeStruct((ND + 16,), jnp.int32),
    ),
    mesh=_mesh,
    scratch_types=[
        pltpu.VMEM((ND + 16,), jnp.int32),
        pltpu.VMEM((NT * 16,), jnp.int32),
        pltpu.VMEM((48,), jnp.int32),
        pltpu.VMEM((ECPAD,), jnp.int32),
        pltpu.VMEM((ECPAD,), jnp.int32),
        pltpu.VMEM((NCHUNK, K), jnp.int32),
        pltpu.VMEM((NCHUNK, K), jnp.int32),
        pltpu.VMEM((K,), jnp.int32),
        pltpu.SemaphoreType.DMA,
    ],
)(_b3_body)


def _c_body(h_hbm, binned_hbm, sg_hbm, mx_hbm, mn_hbm,
            accx, accn, rows_v, vals_v, s_v, sem):
    t = _wid()
    pltpu.sync_copy(sg_hbm.at[pl.ds(pl.multiple_of(t * RNG, 8), RNG + 16)], s_v)
    zz = jnp.zeros((16,), jnp.float32)
    big = jnp.zeros((16,), jnp.float32) + 1e30
    NC16 = D // 16

    for g in range(2):
        row0 = (2 * t + g) * GR
        sbase = g * GR
        e_lo = _sread(s_v, sbase)
        e_hi = _sread(s_v, sbase + GR)

        def zinit(r, c):
            for j in range(NC16):
                sl = pl.ds(j * 16, 16)
                accx[r, sl] = zz
                accn[r, sl] = big
            return c
        lax.fori_loop(0, GR, zinit, 0)

        a_lo = pl.multiple_of((e_lo >> 3) << 3, 8)
        nb = (e_hi - a_lo + (K - 1)) >> 7

        def batch(bi, r_carry, sbase=sbase):
            b0 = a_lo + bi * K
            b1e = b0 + K
            pltpu.sync_copy(binned_hbm.at[pl.ds(pl.multiple_of(b0, 8), K)],
                            vals_v.at[pl.ds(0, K)])
            pltpu.async_copy(h_hbm.at[vals_v.at[pl.ds(0, K)]], rows_v,
                             sem).wait()

            def wcond(st):
                r, cont = st
                return (cont == 1) & (r < GR)

            def wbody(st, sbase=sbase, b0=b0, b1e=b1e):
                r, _ = st
                s_lo = _sread(s_v, sbase + r)
                s_hi = _sread(s_v, sbase + r + 1)
                e0 = jnp.maximum(s_lo, b0)
                e1 = jnp.minimum(s_hi, b1e)
                acc = tuple(
                    [accx[r, pl.ds(j * 16, 16)] for j in range(NC16)]
                    + [accn[r, pl.ds(j * 16, 16)] for j in range(NC16)]
                )

                def edge(e, a, b0=b0):
                    k = e - b0
                    out = []
                    for j in range(NC16):
                        rv = rows_v[k, pl.ds(j * 16, 16)]
                        out.append(jnp.maximum(a[j], rv))
                    for j in range(NC16):
                        rv = rows_v[k, pl.ds(j * 16, 16)]
                        out.append(jnp.minimum(a[NC16 + j], rv))
                    return tuple(out)
                acc = lax.fori_loop(e0, e1, edge, acc)
                for j in range(NC16):
                    accx[r, pl.ds(j * 16, 16)] = acc[j]
                    accn[r, pl.ds(j * 16, 16)] = acc[NC16 + j]
                adv = (s_hi <= b1e).astype(jnp.int32)
                return (r + adv, adv)

            r_carry, _ = lax.while_loop(wcond, wbody, (r_carry, jnp.int32(1)))
            return r_carry
        lax.fori_loop(0, nb, batch, jnp.int32(0))

        row0a = pl.multiple_of(row0, 8)

        @pl.when(row0 + GR <= N)
        def _():
            pltpu.sync_copy(accx, mx_hbm.at[pl.ds(row0a, GR)])
            pltpu.sync_copy(accn, mn_hbm.at[pl.ds(row0a, GR)])

        @pl.when((row0 < N) & (row0 + GR > N))
        def _():
            pltpu.sync_copy(accx.at[pl.ds(0, N % GR)],
                            mx_hbm.at[pl.ds(row0a, N % GR)])
            pltpu.sync_copy(accn.at[pl.ds(0, N % GR)],
                            mn_hbm.at[pl.ds(row0a, N % GR)])


_c_call = functools.partial(
    pl.kernel,
    compiler_params=pltpu.CompilerParams(needs_layout_passes=False),
    out_type=(
        jax.ShapeDtypeStruct((N, D), jnp.float32),
        jax.ShapeDtypeStruct((N, D), jnp.float32),
    ),
    mesh=_mesh,
    scratch_types=[
        pltpu.VMEM((GR, D), jnp.float32),
        pltpu.VMEM((GR, D), jnp.float32),
        pltpu.VMEM((K, D), jnp.float32),
        pltpu.VMEM((K + 16,), jnp.int32),
        pltpu.VMEM((RNG + 16,), jnp.int32),
        pltpu.SemaphoreType.DMA,
    ],
)(_c_body)


def kernel(in_feat, edge_index, W1_pool, b1_pool, W1_self, W1_eve, pw1, bias1,
           W2_pool, b2_pool, W2_self, W2_eve, pw2, bias2):
    src = edge_index[0].astype(jnp.int32)
    dst = edge_index[1].astype(jnp.int32)

    hist = _b1(dst)
    offs_partial, rtot = _b2(hist)
    binned, sg = _b3(src, dst, offs_partial, rtot)

    def layer(x, W_pool, b_pool, W_self, W_eve, pw, bias, relu):
        h = _pool_matmul(x, W_pool, b_pool)
        mx, mn = _c_call(h, binned, sg)
        return _combine(x, mx, mn, W_self, pw[0] * W_eve, pw[1] * W_eve,
                        bias, relu)

    h1 = layer(in_feat, W1_pool, b1_pool, W1_self, W1_eve, pw1, bias1, True)
    out = layer(h1, W2_pool, b2_pool, W2_self, W2_eve, pw2, bias2, False)
    return out


# speedup vs baseline: 7.4825x; 1.2198x over previous
import functools

import jax
import jax.numpy as jnp
from jax import lax
from jax.experimental import pallas as pl
from jax.experimental.pallas import tpu as pltpu
from jax.experimental.pallas import tpu_sc as plsc

N = 10000
E = 160000
D = 256

NT = 32
EC = E // NT
RNG = 320
ND = NT * RNG
GR = 160
GR2 = 80
K = 128
NCHUNK = (EC + K - 1) // K
ECPAD = NCHUNK * K
BINCAP = E + 4096

_BLK = 1000

_mesh = plsc.VectorSubcoreMesh(core_axis_name="c", subcore_axis_name="s")


def _wid():
    return lax.axis_index("s") * 2 + lax.axis_index("c")


def _sread(ref, i):
    return ref[pl.ds(i, 16)][0]


def _runinfo(sd):
    iota = lax.iota(jnp.int32, 16)
    prev = sd.at[jnp.maximum(iota - 1, 0)].get(mode="promise_in_bounds")
    first = (sd != prev) | (iota == 0)
    runstart = plsc.cummax(jnp.where(first, iota, 0))
    rank = iota - runstart
    nfirst = first.astype(jnp.int32).at[
        jnp.minimum(iota + 1, 15)].get(mode="promise_in_bounds")
    last = (iota == 15) | (nfirst == 1)
    return rank, last


def _pool_body(x_ref, w_ref, b_ref, o_ref):
    acc = jnp.dot(x_ref[...], w_ref[...].T, preferred_element_type=jnp.float32)
    o_ref[...] = jnp.maximum(acc + b_ref[...][None, :], 0.0)


def _pool_matmul(x, w, b):
    return pl.pallas_call(
        _pool_body,
        grid=(N // _BLK,),
        in_specs=[
            pl.BlockSpec((_BLK, D), lambda i: (i, 0)),
            pl.BlockSpec((D, D), lambda i: (0, 0)),
            pl.BlockSpec((D,), lambda i: (0,)),
        ],
        out_specs=pl.BlockSpec((_BLK, D), lambda i: (i, 0)),
        out_shape=jax.ShapeDtypeStruct((N, D), jnp.float32),
    )(x, w, b)


def _combine_body(x_ref, mx_ref, mn_ref, ws_ref, we0_ref, we1_ref, b_ref,
                  o_ref, *, relu):
    mn = mn_ref[...]
    mn = jnp.where(mn >= 1e30, 0.0, mn)
    acc = jnp.dot(x_ref[...], ws_ref[...].T, preferred_element_type=jnp.float32)
    acc += jnp.dot(mx_ref[...], we0_ref[...].T, preferred_element_type=jnp.float32)
    acc += jnp.dot(mn, we1_ref[...].T, preferred_element_type=jnp.float32)
    acc += b_ref[...][None, :]
    if relu:
        acc = jnp.maximum(acc, 0.0)
    o_ref[...] = acc


def _combine(x, mx, mn, w_self, we0, we1, bias, relu):
    return pl.pallas_call(
        functools.partial(_combine_body, relu=relu),
        grid=(N // _BLK,),
        in_specs=[
            pl.BlockSpec((_BLK, D), lambda i: (i, 0)),
            pl.BlockSpec((_BLK, D), lambda i: (i, 0)),
            pl.BlockSpec((_BLK, D), lambda i: (i, 0)),
            pl.BlockSpec((D, D), lambda i: (0, 0)),
            pl.BlockSpec((D, D), lambda i: (0, 0)),
            pl.BlockSpec((D, D), lambda i: (0, 0)),
            pl.BlockSpec((D,), lambda i: (0,)),
        ],
        out_specs=pl.BlockSpec((_BLK, D), lambda i: (i, 0)),
        out_shape=jax.ShapeDtypeStruct((N, D), jnp.float32),
    )(x, mx, mn, w_self, we0, we1, bias)


def _b1_body(dst_hbm, hist_hbm, dst_v, hist_v):
    t = _wid()
    zero = jnp.zeros((16,), jnp.int32)
    iota = lax.iota(jnp.int32, 16)

    def zinit(i, c):
        hist_v[pl.ds(i * 16, 16)] = zero
        return c
    lax.fori_loop(0, ND // 16, zinit, 0)

    pltpu.sync_copy(dst_hbm.at[pl.ds(pl.multiple_of(t * EC, 8), EC)],
                    dst_v.at[pl.ds(0, EC)])

    def cnt(i, c):
        base = i * 16
        d = dst_v[pl.ds(base, 16)]
        valid = (base + iota) < EC
        d = jnp.where(valid, d, ND - 1)
        sd, _ = plsc.sort_key_val(d, d)
        rank, last = _runinfo(sd)
        old = plsc.load_gather(hist_v, [sd])
        plsc.store_scatter(hist_v, [sd], old + rank + 1,
                           mask=last & (sd != ND - 1))
        return c
    lax.fori_loop(0, (EC + 15) // 16, cnt, 0)

    pltpu.sync_copy(hist_v, hist_hbm.at[pl.ds(pl.multiple_of(t * ND, 8), ND)])


_b1 = functools.partial(
    pl.kernel,
    compiler_params=pltpu.CompilerParams(needs_layout_passes=False),
    out_type=jax.ShapeDtypeStruct((NT * ND,), jnp.int32),
    mesh=_mesh,
    scratch_types=[
        pltpu.VMEM((ECPAD,), jnp.int32),
        pltpu.VMEM((ND,), jnp.int32),
    ],
)(_b1_body)


def _b2_body(hist_hbm, offs_hbm, rtot_hbm, cols, pre, sloc, rt_row):
    u = _wid()
    lo = u * RNG

    def load_t(t, c):
        pltpu.sync_copy(hist_hbm.at[pl.ds(pl.multiple_of(t * ND + lo, 8), RNG)],
                        cols.at[pl.ds(pl.multiple_of(t * RNG, 8), RNG)])
        return c
    lax.fori_loop(0, NT, load_t, 0)

    for j in range(RNG // 16):
        sl = pl.ds(j * 16, 16)

        def body_t(t, acc, j=j):
            pre[pl.ds(t * RNG + j * 16, 16)] = acc
            return acc + cols[pl.ds(t * RNG + j * 16, 16)]
        tot = lax.fori_loop(0, NT, body_t, jnp.zeros((16,), jnp.int32))
        sloc[sl] = tot

    carry = jnp.int32(0)
    for j in range(RNG // 16):
        sl = pl.ds(j * 16, 16)
        v = sloc[sl]
        incl = plsc.cumsum(v)
        sloc[sl] = incl - v + carry
        carry = carry + jnp.sum(v)

    def out_t(t, c):
        for j in range(RNG // 16):
            sl = pl.ds(j * 16, 16)
            o = pl.ds(t * RNG + j * 16, 16)
            pre[o] = pre[o] + sloc[sl]
        pltpu.sync_copy(pre.at[pl.ds(pl.multiple_of(t * RNG, 8), RNG)],
                        offs_hbm.at[pl.ds(pl.multiple_of(t * ND + lo, 8), RNG)])
        return c
    lax.fori_loop(0, NT, out_t, 0)

    rt_row[...] = jnp.zeros((16,), jnp.int32) + carry
    pltpu.sync_copy(rt_row, rtot_hbm.at[pl.ds(pl.multiple_of(u * 16, 8), 16)])


_b2 = functools.partial(
    pl.kernel,
    compiler_params=pltpu.CompilerParams(needs_layout_passes=False),
    out_type=(
        jax.ShapeDtypeStruct((NT * ND,), jnp.int32),
        jax.ShapeDtypeStruct((NT * 16,), jnp.int32),
    ),
    mesh=_mesh,
    scratch_types=[
        pltpu.VMEM((NT * RNG,), jnp.int32),
        pltpu.VMEM((NT * RNG,), jnp.int32),
        pltpu.VMEM((RNG,), jnp.int32),
        pltpu.VMEM((16,), jnp.int32),
    ],
)(_b2_body)


def _b3_body(src_hbm, dst_hbm, offs_hbm, rtot_hbm, binned_hbm, sg_hbm,
             offs_v, rt_v, base_v, src_v, dst_v, pos_v, dat_v, zpad_v, sem):
    t = _wid()
    iota = lax.iota(jnp.int32, 16)
    pltpu.sync_copy(offs_hbm.at[pl.ds(pl.multiple_of(t * ND, 8), ND)],
                    offs_v.at[pl.ds(0, ND)])
    pltpu.sync_copy(rtot_hbm, rt_v)

    t0 = plsc.load_gather(rt_v, [iota * 16])
    t1 = plsc.load_gather(rt_v, [(iota + 16) * 16])
    base_v[pl.ds(0, 16)] = plsc.cumsum(t0) - t0
    base_v[pl.ds(16, 16)] = plsc.cumsum(t1) - t1 + jnp.sum(t0)

    def addb(u, c):
        b = _sread(base_v, u)
        bv = jnp.zeros((16,), jnp.int32) + b
        for j in range(RNG // 16):
            sl = pl.ds(u * RNG + j * 16, 16)
            offs_v[sl] = offs_v[sl] + bv
        return c
    lax.fori_loop(0, NT, addb, 0)

    @pl.when(t == 0)
    def _():
        offs_v[pl.ds(ND, 16)] = jnp.zeros((16,), jnp.int32) + E
        pltpu.sync_copy(offs_v, sg_hbm)

    pltpu.sync_copy(src_hbm.at[pl.ds(pl.multiple_of(t * EC, 8), EC)],
                    src_v.at[pl.ds(0, EC)])
    pltpu.sync_copy(dst_hbm.at[pl.ds(pl.multiple_of(t * EC, 8), EC)],
                    dst_v.at[pl.ds(0, EC)])

    def emit_ch(ch, c):
        for jj in range(K // 16):
            base = (ch * (K // 16) + jj) * 16
            d = dst_v[pl.ds(base, 16)]
            s = src_v[pl.ds(base, 16)]
            valid = (base + iota) < EC
            d = jnp.where(valid, d, ND - 1)
            sd, ss = plsc.sort_key_val(d, s)
            rank, last = _runinfo(sd)
            inval = sd == ND - 1
            old = plsc.load_gather(offs_v, [sd])
            plsc.store_scatter(offs_v, [sd], old + rank + 1,
                               mask=last & ~inval)
            pos = jnp.where(inval, E + t * K + jj * 16 + iota, old + rank)
            pos_v[ch, pl.ds(jj * 16, 16)] = pos
            dat_v[ch, pl.ds(jj * 16, 16)] = jnp.where(inval, 0, ss)
        return c
    lax.fori_loop(0, NCHUNK, emit_ch, 0)

    for jj in range(K // 16):
        zpad_v[pl.ds(jj * 16, 16)] = jnp.zeros((16,), jnp.int32)
    pltpu.sync_copy(zpad_v, binned_hbm.at[pl.ds(pl.multiple_of(E + t * K, 8), K)])

    descs = [
        pltpu.async_copy(dat_v.at[ch], binned_hbm.at[pos_v.at[ch]], sem)
        for ch in range(NCHUNK)
    ]
    for dsc in descs:
        dsc.wait()


_b3 = functools.partial(
    pl.kernel,
    compiler_params=pltpu.CompilerParams(needs_layout_passes=False),
    out_type=(
        jax.ShapeDtypeStruct((BINCAP,), jnp.int32),
        jax.ShapeDtypeStruct((ND + 16,), jnp.int32),
    ),
    mesh=_mesh,
    scratch_types=[
        pltpu.VMEM((ND + 16,), jnp.int32),
        pltpu.VMEM((NT * 16,), jnp.int32),
        pltpu.VMEM((48,), jnp.int32),
        pltpu.VMEM((ECPAD,), jnp.int32),
        pltpu.VMEM((ECPAD,), jnp.int32),
        pltpu.VMEM((NCHUNK, K), jnp.int32),
        pltpu.VMEM((NCHUNK, K), jnp.int32),
        pltpu.VMEM((K,), jnp.int32),
        pltpu.SemaphoreType.DMA,
    ],
)(_b3_body)


def _c_body(h_hbm, binned_hbm, sg_hbm, mx_hbm, mn_hbm,
            accx, accn, rows0, rows1, vals0, vals1, s_v,
            semv0, semv1, semg0, semg1):
    t = _wid()
    pltpu.sync_copy(sg_hbm.at[pl.ds(pl.multiple_of(t * RNG, 8), RNG + 16)], s_v)
    rows = (rows0, rows1)
    vals = (vals0, vals1)
    semv = (semv0, semv1)
    semg = (semg0, semg1)

    for g in range(RNG // GR2):
        row0 = ((RNG // GR2) * t + g) * GR2
        sbase = g * GR2
        e_lo = _sread(s_v, sbase)
        e_hi = _sread(s_v, sbase + GR2)

        @pl.when(row0 < N)
        def _(row0=row0, sbase=sbase, e_lo=e_lo, e_hi=e_hi):
            _c_group(h_hbm, binned_hbm, mx_hbm, mn_hbm, accx, accn,
                     rows, vals, semv, semg, s_v, row0, sbase, e_lo, e_hi)


def _c_group(h_hbm, binned_hbm, mx_hbm, mn_hbm, accx, accn,
             rows, vals, semv, semg, s_v, row0, sbase, e_lo, e_hi):
        zz = jnp.zeros((16,), jnp.float32)
        big = jnp.zeros((16,), jnp.float32) + 1e30
        NC16 = D // 16

        def fire_vals(b, b0):
            pltpu.async_copy(binned_hbm.at[pl.ds(pl.multiple_of(b0, 8), K)],
                             vals[b].at[pl.ds(0, K)], semv[b])

        def wait_vals(b):
            pltpu.make_async_copy(binned_hbm.at[pl.ds(0, K)],
                                  vals[b].at[pl.ds(0, K)], semv[b]).wait()

        def fire_gather(b):
            pltpu.async_copy(h_hbm.at[vals[b].at[pl.ds(0, K)]], rows[b],
                             semg[b])

        def wait_gather(b):
            pltpu.make_async_copy(h_hbm.at[vals[b].at[pl.ds(0, K)]], rows[b],
                                  semg[b]).wait()

        def zinit(r, c):
            for j in range(NC16):
                sl = pl.ds(j * 16, 16)
                accx[r, sl] = zz
                accn[r, sl] = big
            return c
        lax.fori_loop(0, GR2, zinit, 0)

        a_lo = pl.multiple_of((e_lo >> 3) << 3, 8)
        nb = (e_hi - a_lo + (K - 1)) >> 7

        fire_vals(0, a_lo)
        wait_vals(0)
        fire_gather(0)

        @pl.when(nb > 1)
        def _():
            fire_vals(1, a_lo + K)

        def pair(j2, r_carry, sbase=sbase, a_lo=a_lo, nb=nb, e_lo=e_lo,
                 e_hi=e_hi):
            for b in range(2):
                j = j2 * 2 + b
                b0 = a_lo + j * K
                b1e = b0 + K

                @pl.when(j + 1 < nb)
                def _(b=b):
                    wait_vals(b ^ 1)
                    fire_gather(b ^ 1)

                @pl.when(j < nb)
                def _(b=b):
                    wait_gather(b)

                @pl.when(j + 2 < nb)
                def _(b=b, b0=b0):
                    fire_vals(b, b0 + 2 * K)

                rows_v = rows[b]

                def wcond(st):
                    r, cont = st
                    return (cont == 1) & (r < GR2)

                def wbody(st, sbase=sbase, b0=b0, b1e=b1e, rows_v=rows_v):
                    r, _ = st
                    s_lo = _sread(s_v, sbase + r)
                    s_hi = _sread(s_v, sbase + r + 1)
                    e0 = jnp.maximum(s_lo, b0)
                    e1 = jnp.minimum(s_hi, b1e)
                    acc = tuple(
                        [accx[r, pl.ds(j * 16, 16)] for j in range(NC16)]
                        + [accn[r, pl.ds(j * 16, 16)] for j in range(NC16)]
                    )

                    def edge(e, a, b0=b0, rows_v=rows_v):
                        k = e - b0
                        out = []
                        for j in range(NC16):
                            rv = rows_v[k, pl.ds(j * 16, 16)]
                            out.append(jnp.maximum(a[j], rv))
                        for j in range(NC16):
                            rv = rows_v[k, pl.ds(j * 16, 16)]
                            out.append(jnp.minimum(a[NC16 + j], rv))
                        return tuple(out)
                    acc = lax.fori_loop(e0, e1, edge, acc)
                    for j in range(NC16):
                        accx[r, pl.ds(j * 16, 16)] = acc[j]
                        accn[r, pl.ds(j * 16, 16)] = acc[NC16 + j]
                    adv = (s_hi <= b1e).astype(jnp.int32)
                    return (r + adv, adv)

                r_carry, _ = lax.while_loop(wcond, wbody,
                                            (r_carry, jnp.int32(1)))
            return r_carry
        lax.fori_loop(0, (nb + 1) >> 1, pair, jnp.int32(0))

        row0a = pl.multiple_of(row0, 8)
        pltpu.sync_copy(accx, mx_hbm.at[pl.ds(row0a, GR2)])
        pltpu.sync_copy(accn, mn_hbm.at[pl.ds(row0a, GR2)])
        pltpu.sync_copy(mx_hbm.at[pl.ds(row0a, GR2)], rows[0].at[pl.ds(0, GR2)])
        pltpu.sync_copy(mn_hbm.at[pl.ds(row0a, GR2)], rows[1].at[pl.ds(0, GR2)])


_c_call = functools.partial(
    pl.kernel,
    compiler_params=pltpu.CompilerParams(needs_layout_passes=False),
    out_type=(
        jax.ShapeDtypeStruct((N, D), jnp.float32),
        jax.ShapeDtypeStruct((N, D), jnp.float32),
    ),
    mesh=_mesh,
    scratch_types=[
        pltpu.VMEM((GR2, D), jnp.float32),
        pltpu.VMEM((GR2, D), jnp.float32),
        pltpu.VMEM((K, D), jnp.float32),
        pltpu.VMEM((K, D), jnp.float32),
        pltpu.VMEM((K + 16,), jnp.int32),
        pltpu.VMEM((K + 16,), jnp.int32),
        pltpu.VMEM((RNG + 16,), jnp.int32),
        pltpu.SemaphoreType.DMA,
        pltpu.SemaphoreType.DMA,
        pltpu.SemaphoreType.DMA,
        pltpu.SemaphoreType.DMA,
    ],
)(_c_body)


def kernel(in_feat, edge_index, W1_pool, b1_pool, W1_self, W1_eve, pw1, bias1,
           W2_pool, b2_pool, W2_self, W2_eve, pw2, bias2):
    src = edge_index[0].astype(jnp.int32)
    dst = edge_index[1].astype(jnp.int32)

    hist = _b1(dst)
    offs_partial, rtot = _b2(hist)
    binned, sg = _b3(src, dst, offs_partial, rtot)

    def layer(x, W_pool, b_pool, W_self, W_eve, pw, bias, relu):
        h = _pool_matmul(x, W_pool, b_pool)
        mx, mn = _c_call(h, binned, sg)
        return _combine(x, mx, mn, W_self, pw[0] * W_eve, pw[1] * W_eve,
                        bias, relu)

    h1 = layer(in_feat, W1_pool, b1_pool, W1_self, W1_eve, pw1, bias1, True)
    out = layer(h1, W2_pool, b2_pool, W2_self, W2_eve, pw2, bias2, False)
    return out
